# baseline (speedup 1.0000x reference)
import jax
import jax.numpy as jnp
from jax.experimental import pallas as pl
from jax.experimental.pallas import tpu as pltpu

_N = 10000
_BM = 400
_NB = _N // _BM


def _gcn_body(x_ref, adj_ref, w1_ref, b1_ref, w2_ref, b2_ref, out_ref,
              s1_ref, s2_ref):
    p = pl.program_id(0)
    i = pl.program_id(1)

    @pl.when((p == 0) & (i == 0))
    def _():
        s1_ref[...] = jnp.dot(x_ref[...], w1_ref[...],
                              preferred_element_type=jnp.float32)

    @pl.when(p == 0)
    def _():
        h = jnp.dot(adj_ref[...], s1_ref[...],
                    preferred_element_type=jnp.float32) + b1_ref[...]
        h = jnp.maximum(h, 0.0)
        s2_ref[pl.ds(i * _BM, _BM), :] = jnp.dot(
            h, w2_ref[...], preferred_element_type=jnp.float32)

    @pl.when(p == 1)
    def _():
        out_ref[...] = jnp.dot(adj_ref[...], s2_ref[...],
                               preferred_element_type=jnp.float32) + b2_ref[...]


def kernel(x, adj, W1, b1, W2, b2):
    n, nfeat = x.shape
    nhid = W1.shape[1]
    nclass = W2.shape[1]
    b1r = b1.reshape(1, nhid)
    b2r = b2.reshape(1, nclass)

    grid = (2, _NB)

    def adj_map(p, i):
        return (jnp.where(p == 0, i, _NB - 1 - i), 0)

    def out_map(p, i):
        return (jnp.where(p == 0, _NB - 1, _NB - 1 - i), 0)

    return pl.pallas_call(
        _gcn_body,
        grid=grid,
        in_specs=[
            pl.BlockSpec((n, nfeat), lambda p, i: (0, 0)),
            pl.BlockSpec((_BM, n), adj_map),
            pl.BlockSpec((nfeat, nhid), lambda p, i: (0, 0)),
            pl.BlockSpec((1, nhid), lambda p, i: (0, 0)),
            pl.BlockSpec((nhid, nclass), lambda p, i: (0, 0)),
            pl.BlockSpec((1, nclass), lambda p, i: (0, 0)),
        ],
        out_specs=pl.BlockSpec((_BM, nclass), out_map),
        out_shape=jax.ShapeDtypeStruct((n, nclass), jnp.float32),
        scratch_shapes=[
            pltpu.VMEM((n, nhid), jnp.float32),
            pltpu.VMEM((n, nclass), jnp.float32),
        ],
    )(x, adj, W1, b1r, W2, b2r)
